# baseline (device time: 77300 ns/iter reference)
import jax
import jax.numpy as jnp
from jax import lax
from jax.experimental import pallas as pl
from jax.experimental.pallas import tpu as pltpu

N_DEV = 8
B, S, H, Dh, Dr = 2, 256, 16, 64, 32
D = 1024
DC = 64
BS = B * S
SCALE = (Dh + Dr) ** -0.5
BF = jnp.bfloat16
F32 = jnp.float32

N_BUF = 3
N_HOP = N_DEV - 1


def kernel(x, Wdkv, Wuk, Wuv, Wq, Wqr, Wkr, Wo):
    x2 = x.reshape(BS, D)

    def body(x_ref, wdkv_ref, wuk_ref, wuv_ref, wq_ref, wqr_ref, wkr_ref,
             wo_ref, out_ref,
             c_buf, uk_buf, uv_buf, kacc, vacc, kb_buf, vb_buf,
             q_buf, qr_buf, kr_buf, o_buf, send_sems, recv_sems):
        me = lax.axis_index("i")
        left = lax.rem(me + N_DEV - 1, N_DEV)
        right = lax.rem(me + 1, N_DEV)

        barrier_sem = pltpu.get_barrier_semaphore()
        for nbr in (left, right):
            pl.semaphore_signal(
                barrier_sem, inc=1,
                device_id=(nbr,), device_id_type=pl.DeviceIdType.MESH,
            )
        pl.semaphore_wait(barrier_sem, 2)

        xb = x_ref[...].astype(BF)

        c_buf[0] = jnp.dot(xb, wdkv_ref[...].astype(BF),
                           preferred_element_type=F32).astype(BF)
        uk_buf[0] = wuk_ref[...].astype(BF)
        uv_buf[0] = wuv_ref[...].astype(BF)

        def accum(s):
            ck = c_buf[s]
            kp = jnp.dot(ck, uk_buf[s], preferred_element_type=F32)
            vp = jnp.dot(ck, uv_buf[s], preferred_element_type=F32)
            if s == 0:
                kacc[...] = kp
                vacc[...] = vp
            else:
                kacc[...] = kacc[...] + kp
                vacc[...] = vacc[...] + vp

        for h in range(N_HOP):
            rdmas = []
            for bi, buf in enumerate((c_buf, uk_buf, uv_buf)):
                r = pltpu.make_async_remote_copy(
                    src_ref=buf.at[h],
                    dst_ref=buf.at[h + 1],
                    send_sem=send_sems.at[bi, h],
                    recv_sem=recv_sems.at[bi, h],
                    device_id=(right,),
                    device_id_type=pl.DeviceIdType.MESH,
                )
                r.start()
                rdmas.append(r)
            accum(h)
            if h == 0:
                q_buf[...] = jnp.dot(xb, wq_ref[...].astype(BF),
                                     preferred_element_type=F32).astype(BF)
            elif h == 1:
                qr_buf[...] = jnp.dot(xb, wqr_ref[...].astype(BF),
                                      preferred_element_type=F32).astype(BF)
                kr_buf[...] = jnp.dot(xb, wkr_ref[...].astype(BF),
                                      preferred_element_type=F32).astype(BF)
            for r in rdmas:
                r.wait()
        accum(N_HOP)

        kb_buf[...] = kacc[...].astype(BF)
        vb_buf[...] = vacc[...].astype(BF)

        for b in range(B):
            rows = slice(b * S, (b + 1) * S)
            kr = kr_buf[rows, :]
            for hh in range(H):
                dcols = slice(hh * Dh, (hh + 1) * Dh)
                rcols = slice(hh * Dr, (hh + 1) * Dr)
                q = q_buf[rows, dcols]
                k = kb_buf[rows, dcols]
                qr = qr_buf[rows, rcols]
                v = vb_buf[rows, dcols]
                s_nope = lax.dot_general(
                    q, k, (((1,), (1,)), ((), ())),
                    preferred_element_type=F32)
                s_rope = lax.dot_general(
                    qr, kr, (((1,), (1,)), ((), ())),
                    preferred_element_type=F32)
                s = (s_nope + s_rope) * SCALE
                m = jnp.max(s, axis=1, keepdims=True)
                e = jnp.exp(s - m)
                p = (e / jnp.sum(e, axis=1, keepdims=True)).astype(BF)
                o = jnp.dot(p, v, preferred_element_type=F32)
                o_buf[rows, dcols] = o.astype(BF)

        out_ref[...] = jnp.dot(o_buf[...], wo_ref[...].astype(BF),
                               preferred_element_type=F32)

    vmem = pl.BlockSpec(memory_space=pltpu.VMEM)
    out2 = pl.pallas_call(
        body,
        out_shape=jax.ShapeDtypeStruct((BS, D), F32),
        in_specs=[vmem] * 8,
        out_specs=vmem,
        scratch_shapes=[
            pltpu.VMEM((N_DEV, BS, DC), BF),
            pltpu.VMEM((N_DEV, DC, D), BF),
            pltpu.VMEM((N_DEV, DC, D), BF),
            pltpu.VMEM((BS, D), F32),
            pltpu.VMEM((BS, D), F32),
            pltpu.VMEM((BS, D), BF),
            pltpu.VMEM((BS, D), BF),
            pltpu.VMEM((BS, D), BF),
            pltpu.VMEM((BS, H * Dr), BF),
            pltpu.VMEM((BS, Dr), BF),
            pltpu.VMEM((BS, D), BF),
            pltpu.SemaphoreType.DMA((N_BUF, N_HOP)),
            pltpu.SemaphoreType.DMA((N_BUF, N_HOP)),
        ],
        compiler_params=pltpu.CompilerParams(collective_id=0),
    )(x2, Wdkv, Wuk, Wuv, Wq, Wqr, Wkr, Wo)
    return out2.reshape(B, S, D)


# device time: 67365 ns/iter; 1.1475x vs baseline; 1.1475x over previous
import jax
import jax.numpy as jnp
from jax import lax
from jax.experimental import pallas as pl
from jax.experimental.pallas import tpu as pltpu

N_DEV = 8
B, S, H, Dh, Dr = 2, 256, 16, 64, 32
D = 1024
DC = 64
BS = B * S
SCALE = (Dh + Dr) ** -0.5
BF = jnp.bfloat16
F32 = jnp.float32

N_BUF = 3
N_HOP = N_DEV - 1


def kernel(x, Wdkv, Wuk, Wuv, Wq, Wqr, Wkr, Wo):
    x2 = x.reshape(BS, D)

    def body(x_ref, wdkv_ref, wuk_ref, wuv_ref, wq_ref, wqr_ref, wkr_ref,
             wo_ref, out_ref,
             c_buf, uk_buf, uv_buf, kacc, vacc, kb_buf, vb_buf,
             q_buf, qr_buf, kr_buf, o_buf, send_sems, recv_sems):
        me = lax.axis_index("i")

        barrier_sem = pltpu.get_barrier_semaphore()
        for k in range(1, N_DEV):
            pl.semaphore_signal(
                barrier_sem, inc=1,
                device_id=(lax.rem(me + k, N_DEV),),
                device_id_type=pl.DeviceIdType.MESH,
            )
        pl.semaphore_wait(barrier_sem, N_DEV - 1)

        xb = x_ref[...].astype(BF)

        c_buf[0] = jnp.dot(xb, wdkv_ref[...].astype(BF),
                           preferred_element_type=F32).astype(BF)
        uk_buf[0] = wuk_ref[...].astype(BF)
        uv_buf[0] = wuv_ref[...].astype(BF)

        def accum(s):
            ck = c_buf[s]
            kp = jnp.dot(ck, uk_buf[s], preferred_element_type=F32)
            vp = jnp.dot(ck, uv_buf[s], preferred_element_type=F32)
            if s == 0:
                kacc[...] = kp
                vacc[...] = vp
            else:
                kacc[...] = kacc[...] + kp
                vacc[...] = vacc[...] + vp

        sends = []
        for k in range(1, N_DEV):
            dest = lax.rem(me + k, N_DEV)
            slot = N_DEV - k
            for bi, buf in enumerate((c_buf, uk_buf, uv_buf)):
                r = pltpu.make_async_remote_copy(
                    src_ref=buf.at[0],
                    dst_ref=buf.at[slot],
                    send_sem=send_sems.at[bi, k - 1],
                    recv_sem=recv_sems.at[bi, slot - 1],
                    device_id=(dest,),
                    device_id_type=pl.DeviceIdType.MESH,
                )
                r.start()
                sends.append(r)

        accum(0)
        q_buf[...] = jnp.dot(xb, wq_ref[...].astype(BF),
                             preferred_element_type=F32).astype(BF)
        qr_buf[...] = jnp.dot(xb, wqr_ref[...].astype(BF),
                              preferred_element_type=F32).astype(BF)
        kr_buf[...] = jnp.dot(xb, wkr_ref[...].astype(BF),
                              preferred_element_type=F32).astype(BF)

        for s in range(1, N_DEV):
            for bi, buf in enumerate((c_buf, uk_buf, uv_buf)):
                recv = pltpu.make_async_remote_copy(
                    src_ref=buf.at[0],
                    dst_ref=buf.at[s],
                    send_sem=send_sems.at[bi, s - 1],
                    recv_sem=recv_sems.at[bi, s - 1],
                    device_id=(me,),
                    device_id_type=pl.DeviceIdType.MESH,
                )
                recv.wait_recv()
            accum(s)
        for r in sends:
            r.wait_send()

        kb_buf[...] = kacc[...].astype(BF)
        vb_buf[...] = vacc[...].astype(BF)

        for b in range(B):
            rows = slice(b * S, (b + 1) * S)
            kr = kr_buf[rows, :]
            for hh in range(H):
                dcols = slice(hh * Dh, (hh + 1) * Dh)
                rcols = slice(hh * Dr, (hh + 1) * Dr)
                q = q_buf[rows, dcols]
                k = kb_buf[rows, dcols]
                qr = qr_buf[rows, rcols]
                v = vb_buf[rows, dcols]
                s_nope = lax.dot_general(
                    q, k, (((1,), (1,)), ((), ())),
                    preferred_element_type=F32)
                s_rope = lax.dot_general(
                    qr, kr, (((1,), (1,)), ((), ())),
                    preferred_element_type=F32)
                s = (s_nope + s_rope) * SCALE
                m = jnp.max(s, axis=1, keepdims=True)
                e = jnp.exp(s - m)
                p = (e / jnp.sum(e, axis=1, keepdims=True)).astype(BF)
                o = jnp.dot(p, v, preferred_element_type=F32)
                o_buf[rows, dcols] = o.astype(BF)

        out_ref[...] = jnp.dot(o_buf[...], wo_ref[...].astype(BF),
                               preferred_element_type=F32)

    vmem = pl.BlockSpec(memory_space=pltpu.VMEM)
    out2 = pl.pallas_call(
        body,
        out_shape=jax.ShapeDtypeStruct((BS, D), F32),
        in_specs=[vmem] * 8,
        out_specs=vmem,
        scratch_shapes=[
            pltpu.VMEM((N_DEV, BS, DC), BF),
            pltpu.VMEM((N_DEV, DC, D), BF),
            pltpu.VMEM((N_DEV, DC, D), BF),
            pltpu.VMEM((BS, D), F32),
            pltpu.VMEM((BS, D), F32),
            pltpu.VMEM((BS, D), BF),
            pltpu.VMEM((BS, D), BF),
            pltpu.VMEM((BS, D), BF),
            pltpu.VMEM((BS, H * Dr), BF),
            pltpu.VMEM((BS, Dr), BF),
            pltpu.VMEM((BS, D), BF),
            pltpu.SemaphoreType.DMA((N_BUF, N_HOP)),
            pltpu.SemaphoreType.DMA((N_BUF, N_HOP)),
        ],
        compiler_params=pltpu.CompilerParams(collective_id=0),
    )(x2, Wdkv, Wuk, Wuv, Wq, Wqr, Wkr, Wo)
    return out2.reshape(B, S, D)


# device time: 35265 ns/iter; 2.1920x vs baseline; 1.9103x over previous
import jax
import jax.numpy as jnp
from jax import lax
from jax.experimental import pallas as pl
from jax.experimental.pallas import tpu as pltpu

N_DEV = 8
B, S, H, Dh, Dr = 2, 256, 16, 64, 32
D = 1024
DC = 64
BS = B * S
HPD = H // N_DEV
HB = HPD * Dh
RB = HPD * Dr
SCALE = (Dh + Dr) ** -0.5
BF = jnp.bfloat16
F32 = jnp.float32

N_BUF = 4
N_PEER = N_DEV - 1


def kernel(x, Wdkv, Wuk, Wuv, Wq, Wqr, Wkr, Wo):
    me_out = lax.axis_index("i")
    x2 = x.reshape(BS, D)
    wq_m = lax.dynamic_slice_in_dim(Wq, me_out * HB, HB, 1)
    wqr_m = lax.dynamic_slice_in_dim(Wqr, me_out * RB, RB, 1)
    wuk8 = Wuk.reshape(DC, N_DEV, HB).transpose(1, 0, 2)
    wuv8 = Wuv.reshape(DC, N_DEV, HB).transpose(1, 0, 2)
    wo8 = Wo.reshape(N_DEV, HB, D)

    def body(x_ref, wdkv_ref, wuk8_ref, wuv8_ref, wq_ref, wqr_ref, wkr_ref,
             wo8_ref, out_ref,
             c_gat, uk_send, uv_send, uk_gat, uv_gat,
             kacc, vacc, kb_buf, vb_buf, q_buf, qr_buf, kr_buf, o_gat,
             send_sems, recv_sems):
        me = lax.axis_index("i")

        barrier_sem = pltpu.get_barrier_semaphore()
        for k in range(1, N_DEV):
            pl.semaphore_signal(
                barrier_sem, inc=1,
                device_id=(lax.rem(me + k, N_DEV),),
                device_id_type=pl.DeviceIdType.MESH,
            )
        pl.semaphore_wait(barrier_sem, N_PEER)

        xb = x_ref[...].astype(BF)

        c_gat[0] = jnp.dot(xb, wdkv_ref[...].astype(BF),
                           preferred_element_type=F32).astype(BF)
        uk_send[...] = wuk8_ref[...].astype(BF)
        uv_send[...] = wuv8_ref[...].astype(BF)

        sends = []

        def push(src, dst_buf, slot, bi, k, dest):
            r = pltpu.make_async_remote_copy(
                src_ref=src,
                dst_ref=dst_buf.at[slot],
                send_sem=send_sems.at[bi, k - 1],
                recv_sem=recv_sems.at[bi, slot - 1],
                device_id=(dest,),
                device_id_type=pl.DeviceIdType.MESH,
            )
            r.start()
            sends.append(r)

        for k in range(1, N_DEV):
            dest = lax.rem(me + k, N_DEV)
            slot = N_DEV - k
            push(c_gat.at[0], c_gat, slot, 0, k, dest)
            push(uk_send.at[dest], uk_gat, slot, 1, k, dest)
            push(uv_send.at[dest], uv_gat, slot, 2, k, dest)

        q_buf[...] = jnp.dot(xb, wq_ref[...].astype(BF),
                             preferred_element_type=F32).astype(BF)
        qr_buf[...] = jnp.dot(xb, wqr_ref[...].astype(BF),
                              preferred_element_type=F32).astype(BF)
        kr_buf[...] = jnp.dot(xb, wkr_ref[...].astype(BF),
                              preferred_element_type=F32).astype(BF)
        kacc[...] = jnp.dot(c_gat[0], uk_send[me],
                            preferred_element_type=F32)
        vacc[...] = jnp.dot(c_gat[0], uv_send[me],
                            preferred_element_type=F32)

        def wait_slot(bi, buf, s):
            recv = pltpu.make_async_remote_copy(
                src_ref=buf.at[s],
                dst_ref=buf.at[s],
                send_sem=send_sems.at[bi, s - 1],
                recv_sem=recv_sems.at[bi, s - 1],
                device_id=(me,),
                device_id_type=pl.DeviceIdType.MESH,
            )
            recv.wait_recv()

        for s in range(1, N_DEV):
            wait_slot(0, c_gat, s)
            wait_slot(1, uk_gat, s)
            wait_slot(2, uv_gat, s)
            kacc[...] = kacc[...] + jnp.dot(c_gat[s], uk_gat[s],
                                            preferred_element_type=F32)
            vacc[...] = vacc[...] + jnp.dot(c_gat[s], uv_gat[s],
                                            preferred_element_type=F32)

        kb_buf[...] = kacc[...].astype(BF)
        vb_buf[...] = vacc[...].astype(BF)

        for b in range(B):
            rows = slice(b * S, (b + 1) * S)
            kr = kr_buf[rows, :]
            for hh in range(HPD):
                dcols = slice(hh * Dh, (hh + 1) * Dh)
                rcols = slice(hh * Dr, (hh + 1) * Dr)
                q = q_buf[rows, dcols]
                k = kb_buf[rows, dcols]
                qr = qr_buf[rows, rcols]
                v = vb_buf[rows, dcols]
                s_nope = lax.dot_general(
                    q, k, (((1,), (1,)), ((), ())),
                    preferred_element_type=F32)
                s_rope = lax.dot_general(
                    qr, kr, (((1,), (1,)), ((), ())),
                    preferred_element_type=F32)
                sc = (s_nope + s_rope) * SCALE
                m = jnp.max(sc, axis=1, keepdims=True)
                e = jnp.exp(sc - m)
                p = (e / jnp.sum(e, axis=1, keepdims=True)).astype(BF)
                o = jnp.dot(p, v, preferred_element_type=F32)
                o_gat[0, rows, dcols] = o.astype(BF)

        for k in range(1, N_DEV):
            dest = lax.rem(me + k, N_DEV)
            push(o_gat.at[0], o_gat, N_DEV - k, 3, k, dest)
        out_ref[...] = jnp.dot(o_gat[0], wo8_ref[me].astype(BF),
                               preferred_element_type=F32)
        for s in range(1, N_DEV):
            wait_slot(3, o_gat, s)
            j = lax.rem(me + s, N_DEV)
            out_ref[...] = out_ref[...] + jnp.dot(
                o_gat[s], wo8_ref[j].astype(BF),
                preferred_element_type=F32)

        for r in sends:
            r.wait_send()

    vmem = pl.BlockSpec(memory_space=pltpu.VMEM)
    out2 = pl.pallas_call(
        body,
        out_shape=jax.ShapeDtypeStruct((BS, D), F32),
        in_specs=[vmem] * 8,
        out_specs=vmem,
        scratch_shapes=[
            pltpu.VMEM((N_DEV, BS, DC), BF),
            pltpu.VMEM((N_DEV, DC, HB), BF),
            pltpu.VMEM((N_DEV, DC, HB), BF),
            pltpu.VMEM((N_DEV, DC, HB), BF),
            pltpu.VMEM((N_DEV, DC, HB), BF),
            pltpu.VMEM((BS, HB), F32),
            pltpu.VMEM((BS, HB), F32),
            pltpu.VMEM((BS, HB), BF),
            pltpu.VMEM((BS, HB), BF),
            pltpu.VMEM((BS, HB), BF),
            pltpu.VMEM((BS, RB), BF),
            pltpu.VMEM((BS, Dr), BF),
            pltpu.VMEM((N_DEV, BS, HB), BF),
            pltpu.SemaphoreType.DMA((N_BUF, N_PEER)),
            pltpu.SemaphoreType.DMA((N_BUF, N_PEER)),
        ],
        compiler_params=pltpu.CompilerParams(collective_id=0),
    )(x2, Wdkv, wuk8, wuv8, wq_m, wqr_m, Wkr, wo8)
    return out2.reshape(B, S, D)


# device time: 34838 ns/iter; 2.2188x vs baseline; 1.0123x over previous
import jax
import jax.numpy as jnp
from jax import lax
from jax.experimental import pallas as pl
from jax.experimental.pallas import tpu as pltpu

N_DEV = 8
B, S, H, Dh, Dr = 2, 256, 16, 64, 32
D = 1024
DC = 64
DC_TOT = N_DEV * DC
BS = B * S
HPD = H // N_DEV
HB = HPD * Dh
RB = HPD * Dr
SCALE = (Dh + Dr) ** -0.5
BF = jnp.bfloat16
F32 = jnp.float32

N_BUF = 4
N_PEER = N_DEV - 1


def kernel(x, Wdkv, Wuk, Wuv, Wq, Wqr, Wkr, Wo):
    me_out = lax.axis_index("i")
    x2 = x.reshape(BS, D)
    wq_m = lax.dynamic_slice_in_dim(Wq, me_out * HB, HB, 1)
    wqr_m = lax.dynamic_slice_in_dim(Wqr, me_out * RB, RB, 1)
    wo8 = Wo.reshape(N_DEV, HB, D)

    def body(x_ref, wdkv_ref, wuk_ref, wuv_ref, wq_ref, wqr_ref, wkr_ref,
             wo8_ref, out_ref,
             c_loc, c_all, uk_send, uv_send, uk_all, uv_all,
             kb_buf, vb_buf, q_buf, qr_buf, kr_buf, o_gat, wo_bf,
             send_sems, recv_sems, loc_sems):
        me = lax.axis_index("i")

        c_all[...] = jnp.zeros((BS, N_DEV * HB), BF)
        uk_all[...] = jnp.zeros((N_DEV * HB, HB), BF)
        uv_all[...] = jnp.zeros((N_DEV * HB, HB), BF)

        barrier_sem = pltpu.get_barrier_semaphore()
        for k in range(1, N_DEV):
            pl.semaphore_signal(
                barrier_sem, inc=1,
                device_id=(lax.rem(me + k, N_DEV),),
                device_id_type=pl.DeviceIdType.MESH,
            )
        pl.semaphore_wait(barrier_sem, N_PEER)

        xb = x_ref[...].astype(BF)

        c_loc[...] = jnp.zeros((BS, HB), BF)
        c_loc[:, 0:DC] = jnp.dot(xb, wdkv_ref[...].astype(BF),
                                 preferred_element_type=F32).astype(BF)
        for d in range(N_DEV):
            uk_send[d] = wuk_ref[:, d * HB:(d + 1) * HB].astype(BF)
            uv_send[d] = wuv_ref[:, d * HB:(d + 1) * HB].astype(BF)

        locals_ = [
            pltpu.make_async_copy(
                c_loc, c_all.at[:, pl.ds(me * HB, HB)], loc_sems.at[0]),
            pltpu.make_async_copy(
                uk_send.at[me], uk_all.at[pl.ds(me * HB, DC), :],
                loc_sems.at[1]),
            pltpu.make_async_copy(
                uv_send.at[me], uv_all.at[pl.ds(me * HB, DC), :],
                loc_sems.at[2]),
        ]
        for cp in locals_:
            cp.start()

        sends = []

        def push(src, dst, bi, k, dest):
            r = pltpu.make_async_remote_copy(
                src_ref=src,
                dst_ref=dst,
                send_sem=send_sems.at[bi, k - 1],
                recv_sem=recv_sems.at[bi, N_DEV - k - 1],
                device_id=(dest,),
                device_id_type=pl.DeviceIdType.MESH,
            )
            r.start()
            sends.append(r)

        for k in range(1, N_DEV):
            dest = lax.rem(me + k, N_DEV)
            push(c_loc, c_all.at[:, pl.ds(me * HB, HB)], 0, k, dest)
            push(uk_send.at[dest], uk_all.at[pl.ds(me * HB, DC), :],
                 1, k, dest)
            push(uv_send.at[dest], uv_all.at[pl.ds(me * HB, DC), :],
                 2, k, dest)

        q_buf[...] = jnp.dot(xb, wq_ref[...].astype(BF),
                             preferred_element_type=F32).astype(BF)
        qr_buf[...] = jnp.dot(xb, wqr_ref[...].astype(BF),
                              preferred_element_type=F32).astype(BF)
        kr_buf[...] = jnp.dot(xb, wkr_ref[...].astype(BF),
                              preferred_element_type=F32).astype(BF)
        wo_bf[...] = wo8_ref[...].astype(BF)

        def wait_recv(bi, s, dst):
            recv = pltpu.make_async_remote_copy(
                src_ref=dst,
                dst_ref=dst,
                send_sem=send_sems.at[bi, s - 1],
                recv_sem=recv_sems.at[bi, s - 1],
                device_id=(me,),
                device_id_type=pl.DeviceIdType.MESH,
            )
            recv.wait_recv()

        for cp in locals_:
            cp.wait()
        for s in range(1, N_DEV):
            wait_recv(0, s, c_all.at[:, 0:HB])
            wait_recv(1, s, uk_all.at[0:DC, :])
            wait_recv(2, s, uv_all.at[0:DC, :])

        kb_buf[...] = jnp.dot(c_all[...], uk_all[...],
                              preferred_element_type=F32).astype(BF)
        vb_buf[...] = jnp.dot(c_all[...], uv_all[...],
                              preferred_element_type=F32).astype(BF)

        for b in range(B):
            rows = slice(b * S, (b + 1) * S)
            kr = kr_buf[rows, :]
            for hh in range(HPD):
                dcols = slice(hh * Dh, (hh + 1) * Dh)
                rcols = slice(hh * Dr, (hh + 1) * Dr)
                q = q_buf[rows, dcols]
                k = kb_buf[rows, dcols]
                qr = qr_buf[rows, rcols]
                v = vb_buf[rows, dcols]
                s_nope = lax.dot_general(
                    q, k, (((1,), (1,)), ((), ())),
                    preferred_element_type=F32)
                s_rope = lax.dot_general(
                    qr, kr, (((1,), (1,)), ((), ())),
                    preferred_element_type=F32)
                sc = (s_nope + s_rope) * SCALE
                m = jnp.max(sc, axis=1, keepdims=True)
                e = jnp.exp(sc - m)
                p = (e / jnp.sum(e, axis=1, keepdims=True)).astype(BF)
                o = jnp.dot(p, v, preferred_element_type=F32)
                o_gat[0, rows, dcols] = o.astype(BF)

        for k in range(1, N_DEV):
            dest = lax.rem(me + k, N_DEV)
            push(o_gat.at[0], o_gat.at[N_DEV - k], 3, k, dest)
        out_ref[...] = jnp.dot(o_gat[0], wo_bf[me],
                               preferred_element_type=F32)
        for s in range(1, N_DEV):
            wait_recv(3, s, o_gat.at[s])
            j = lax.rem(me + s, N_DEV)
            out_ref[...] = out_ref[...] + jnp.dot(
                o_gat[s], wo_bf[j], preferred_element_type=F32)

        for r in sends:
            r.wait_send()

    vmem = pl.BlockSpec(memory_space=pltpu.VMEM)
    out2 = pl.pallas_call(
        body,
        out_shape=jax.ShapeDtypeStruct((BS, D), F32),
        in_specs=[vmem] * 8,
        out_specs=vmem,
        scratch_shapes=[
            pltpu.VMEM((BS, HB), BF),
            pltpu.VMEM((BS, N_DEV * HB), BF),
            pltpu.VMEM((N_DEV, DC, HB), BF),
            pltpu.VMEM((N_DEV, DC, HB), BF),
            pltpu.VMEM((N_DEV * HB, HB), BF),
            pltpu.VMEM((N_DEV * HB, HB), BF),
            pltpu.VMEM((BS, HB), BF),
            pltpu.VMEM((BS, HB), BF),
            pltpu.VMEM((BS, HB), BF),
            pltpu.VMEM((BS, RB), BF),
            pltpu.VMEM((BS, Dr), BF),
            pltpu.VMEM((N_DEV, BS, HB), BF),
            pltpu.VMEM((N_DEV, HB, D), BF),
            pltpu.SemaphoreType.DMA((N_BUF, N_PEER)),
            pltpu.SemaphoreType.DMA((N_BUF, N_PEER)),
            pltpu.SemaphoreType.DMA((3,)),
        ],
        compiler_params=pltpu.CompilerParams(collective_id=0),
    )(x2, Wdkv, Wuk, Wuv, wq_m, wqr_m, Wkr, wo8)
    return out2.reshape(B, S, D)
